# baseline (speedup 1.0000x reference)
import functools

import jax
import jax.numpy as jnp
from jax import lax
from jax.experimental import pallas as pl
from jax.experimental.pallas import tpu as pltpu
from jax.experimental.pallas import tpu_sc as plsc

_N = 10000
_D = 128
_E = 320000
_G = 64
_OUT = 10

_NC = 2
_NS = 16
_NW = _NC * _NS
_CH = 128
_EPW = _E // _NW
_NCH = -(-_EPW // _CH)
_EPW_PAD = _NCH * _CH
_NPAD = 10112
_RPS = _NPAD // _NS

_BLK = 10000


def _sc_agg_body(h_hbm, src_hbm, dst_hbm, zero_hbm, out_hbm,
                 src_v, dst_v, rows_v, agg_sh, sem):
    c = lax.axis_index("c")
    s = lax.axis_index("s")
    wid = c * _NS + s
    pltpu.sync_copy(src_hbm.at[wid], src_v)
    pltpu.sync_copy(dst_hbm.at[wid], dst_v)
    pltpu.sync_copy(zero_hbm, agg_sh.at[pl.ds(s * _RPS, _RPS)])
    plsc.subcore_barrier()

    def body(j, carry):
        pltpu.async_copy(h_hbm.at[src_v.at[j]], rows_v, sem).wait()
        pltpu.sync_copy(rows_v, agg_sh.at[dst_v.at[j]], add=True)
        return carry

    lax.fori_loop(0, _NCH, body, 0)
    plsc.subcore_barrier()
    pltpu.sync_copy(agg_sh.at[pl.ds(s * _RPS, _RPS)],
                    out_hbm.at[c].at[pl.ds(s * _RPS, _RPS)])


_sc_agg = pl.kernel(
    _sc_agg_body,
    out_type=jax.ShapeDtypeStruct((_NC, _NPAD, _D), jnp.float32),
    mesh=plsc.VectorSubcoreMesh(core_axis_name="c", subcore_axis_name="s",
                                num_cores=_NC, num_subcores=_NS),
    scratch_types=[
        pltpu.VMEM((_NCH, _CH), jnp.int32),
        pltpu.VMEM((_NCH, _CH), jnp.int32),
        pltpu.VMEM((_CH, _D), jnp.float32),
        pltpu.VMEM_SHARED((_NPAD, _D), jnp.float32),
        pltpu.SemaphoreType.DMA,
    ],
)


def _mm_stats_body(eps_ref, h_ref, a_ref, w_ref, u_ref, s_ref):
    i = pl.program_id(0)
    z = h_ref[...] * eps_ref[0, 0] + a_ref[0] + a_ref[1]
    u = jnp.dot(z, w_ref[...], preferred_element_type=jnp.float32)
    u_ref[...] = u
    st = jnp.concatenate(
        [jnp.sum(u, axis=0, keepdims=True),
         jnp.sum(u * u, axis=0, keepdims=True)], axis=0)

    @pl.when(i == 0)
    def _():
        s_ref[...] = st

    @pl.when(i > 0)
    def _():
        s_ref[...] += st


def _mm_stats(epsb, h, agg, wt):
    return pl.pallas_call(
        _mm_stats_body,
        grid=(_N // _BLK,),
        in_specs=[
            pl.BlockSpec(memory_space=pltpu.SMEM),
            pl.BlockSpec((_BLK, _D), lambda i: (i, 0)),
            pl.BlockSpec((_NC, _BLK, _D), lambda i: (0, i, 0)),
            pl.BlockSpec((_D, _D), lambda i: (0, 0)),
        ],
        out_specs=[
            pl.BlockSpec((_BLK, _D), lambda i: (i, 0)),
            pl.BlockSpec((2, _D), lambda i: (0, 0)),
        ],
        out_shape=[
            jax.ShapeDtypeStruct((_N, _D), jnp.float32),
            jax.ShapeDtypeStruct((2, _D), jnp.float32),
        ],
    )(epsb, h, agg, wt)


def _bn_coeffs(s_ref, g_ref, b_ref):
    mean = s_ref[0:1, :] * (1.0 / _N)
    var = s_ref[1:2, :] * (1.0 / _N) - mean * mean
    inv = lax.rsqrt(var + 1e-5)
    scale = g_ref[...] * inv
    shift = b_ref[...] - mean * scale
    return scale, shift


def _bn_relu_mm_body(s_ref, g_ref, b_ref, u_ref, w_ref, v_ref, s2_ref):
    i = pl.program_id(0)
    scale, shift = _bn_coeffs(s_ref, g_ref, b_ref)
    r = jnp.maximum(u_ref[...] * scale + shift, 0.0)
    v = jnp.dot(r, w_ref[...], preferred_element_type=jnp.float32)
    v_ref[...] = v
    st = jnp.concatenate(
        [jnp.sum(v, axis=0, keepdims=True),
         jnp.sum(v * v, axis=0, keepdims=True)], axis=0)

    @pl.when(i == 0)
    def _():
        s2_ref[...] = st

    @pl.when(i > 0)
    def _():
        s2_ref[...] += st


def _bn_relu_mm(s1, g, b, u, wt):
    return pl.pallas_call(
        _bn_relu_mm_body,
        grid=(_N // _BLK,),
        in_specs=[
            pl.BlockSpec((2, _D), lambda i: (0, 0)),
            pl.BlockSpec((1, _D), lambda i: (0, 0)),
            pl.BlockSpec((1, _D), lambda i: (0, 0)),
            pl.BlockSpec((_BLK, _D), lambda i: (i, 0)),
            pl.BlockSpec((_D, _D), lambda i: (0, 0)),
        ],
        out_specs=[
            pl.BlockSpec((_BLK, _D), lambda i: (i, 0)),
            pl.BlockSpec((2, _D), lambda i: (0, 0)),
        ],
        out_shape=[
            jax.ShapeDtypeStruct((_N, _D), jnp.float32),
            jax.ShapeDtypeStruct((2, _D), jnp.float32),
        ],
    )(s1, g, b, u, wt)


def _bn_relu_body(s_ref, g_ref, b_ref, v_ref, h_ref):
    scale, shift = _bn_coeffs(s_ref, g_ref, b_ref)
    h_ref[...] = jnp.maximum(v_ref[...] * scale + shift, 0.0)


def _bn_relu(s2, g, b, v):
    return pl.pallas_call(
        _bn_relu_body,
        grid=(_N // _BLK,),
        in_specs=[
            pl.BlockSpec((2, _D), lambda i: (0, 0)),
            pl.BlockSpec((1, _D), lambda i: (0, 0)),
            pl.BlockSpec((1, _D), lambda i: (0, 0)),
            pl.BlockSpec((_BLK, _D), lambda i: (i, 0)),
        ],
        out_specs=pl.BlockSpec((_BLK, _D), lambda i: (i, 0)),
        out_shape=jax.ShapeDtypeStruct((_N, _D), jnp.float32),
    )(s2, g, b, v)


def _readout_body(s_ref, g_ref, b_ref, bat_ref, v_ref, wm1_ref, bm1_ref,
                  wm2_ref, bm2_ref, lg_ref, ge_ref, ne_ref,
                  sums_ref, cnts_ref):
    i = pl.program_id(0)

    @pl.when(i == 0)
    def _():
        sums_ref[...] = jnp.zeros_like(sums_ref)
        cnts_ref[...] = jnp.zeros_like(cnts_ref)

    scale, shift = _bn_coeffs(s_ref, g_ref, b_ref)
    h = jnp.maximum(v_ref[...] * scale + shift, 0.0)
    ne_ref[...] = h
    b = bat_ref[0]
    ids = lax.broadcasted_iota(jnp.int32, (_G, _BLK), 0)
    oh = (b == ids).astype(jnp.float32)
    sums_ref[...] += jnp.dot(oh, h, preferred_element_type=jnp.float32)
    cnts_ref[...] += jnp.broadcast_to(
        jnp.sum(oh, axis=1, keepdims=True), (_G, _D))

    @pl.when(i == pl.num_programs(0) - 1)
    def _():
        ge = sums_ref[...] / jnp.maximum(cnts_ref[...], 1.0)
        ge_ref[...] = ge
        z = jnp.dot(ge, wm1_ref[...], preferred_element_type=jnp.float32)
        z = z + bm1_ref[...]
        z = jnp.where(z > 0.0, z, jnp.exp(jnp.minimum(z, 0.0)) - 1.0)
        lg_ref[...] = (jnp.dot(z, wm2_ref[...],
                               preferred_element_type=jnp.float32)
                       + bm2_ref[...])


def _readout(s2, g, b, bat3, v, wm1t, bm1r, wm2tp, bm2p):
    return pl.pallas_call(
        _readout_body,
        grid=(_N // _BLK,),
        in_specs=[
            pl.BlockSpec((2, _D), lambda i: (0, 0)),
            pl.BlockSpec((1, _D), lambda i: (0, 0)),
            pl.BlockSpec((1, _D), lambda i: (0, 0)),
            pl.BlockSpec((1, 1, _BLK), lambda i: (i, 0, 0)),
            pl.BlockSpec((_BLK, _D), lambda i: (i, 0)),
            pl.BlockSpec((_D, _D), lambda i: (0, 0)),
            pl.BlockSpec((1, _D), lambda i: (0, 0)),
            pl.BlockSpec((_D, _D), lambda i: (0, 0)),
            pl.BlockSpec((1, _D), lambda i: (0, 0)),
        ],
        out_specs=[
            pl.BlockSpec((_G, _D), lambda i: (0, 0)),
            pl.BlockSpec((_G, _D), lambda i: (0, 0)),
            pl.BlockSpec((_BLK, _D), lambda i: (i, 0)),
        ],
        out_shape=[
            jax.ShapeDtypeStruct((_G, _D), jnp.float32),
            jax.ShapeDtypeStruct((_G, _D), jnp.float32),
            jax.ShapeDtypeStruct((_N, _D), jnp.float32),
        ],
        scratch_shapes=[
            pltpu.VMEM((_G, _D), jnp.float32),
            pltpu.VMEM((_G, _D), jnp.float32),
        ],
    )(s2, g, b, bat3, v, wm1t, bm1r, wm2tp, bm2p)


def kernel(x, edge_index, batch,
           eps1, W1a, g1a, b1a, W1b, g1b, b1b,
           eps2, W2a, g2a, b2a, W2b, g2b, b2b,
           eps3, W3a, g3a, b3a, W3b, g3b, b3b,
           Wm1, bm1, Wm2, bm2):
    src_w = edge_index[0].reshape(_NW, _EPW)
    dst_w = edge_index[1].reshape(_NW, _EPW)
    pad = _EPW_PAD - _EPW
    src3 = jnp.pad(src_w, ((0, 0), (0, pad))).reshape(_NW, _NCH, _CH)
    dst3 = jnp.pad(dst_w, ((0, 0), (0, pad)),
                   constant_values=_N).reshape(_NW, _NCH, _CH)
    zero_rows = jnp.zeros((_RPS, _D), jnp.float32)

    h = x
    for (eps, Wa, ga, ba, Wb, gb, bb) in (
            (eps1, W1a, g1a, b1a, W1b, g1b, b1b),
            (eps2, W2a, g2a, b2a, W2b, g2b, b2b),
            (eps3, W3a, g3a, b3a, W3b, g3b, b3b)):
        agg = _sc_agg(h, src3, dst3, zero_rows)
        epsb = jnp.reshape(1.0 + eps, (1, 1))
        u, s1 = _mm_stats(epsb, h, agg, Wa.T)
        v, s2 = _bn_relu_mm(s1, ga.reshape(1, _D), ba.reshape(1, _D), u, Wb.T)
        if gb is not g3b:
            h = _bn_relu(s2, gb.reshape(1, _D), bb.reshape(1, _D), v)

    bat3 = batch.reshape(_N // _BLK, 1, _BLK)
    wm2tp = jnp.zeros((_D, _D), jnp.float32).at[:, :_OUT].set(Wm2.T)
    bm2p = jnp.zeros((1, _D), jnp.float32).at[0, :_OUT].set(bm2)
    logits_f, graph_emb, node_emb = _readout(
        s2, g3b.reshape(1, _D), b3b.reshape(1, _D), bat3, v, Wm1.T,
        bm1.reshape(1, _D), wm2tp, bm2p)
    return (logits_f[:, :_OUT], graph_emb, node_emb)


# speedup vs baseline: 1.0071x; 1.0071x over previous
import jax
import jax.numpy as jnp
from jax import lax
from jax.experimental import pallas as pl
from jax.experimental.pallas import tpu as pltpu
from jax.experimental.pallas import tpu_sc as plsc

_N = 10000
_D = 128
_E = 320000
_G = 64
_OUT = 10

_NC = 2
_NS = 16
_NW = _NC * _NS
_CH = 128
_EPW = _E // _NW
_NCH = -(-_EPW // _CH)
_EPW_PAD = _NCH * _CH
_NPAD = 10112
_RPS = _NPAD // _NS

_BLK = 5000


def _sc_agg_body(h_hbm, src_hbm, dst_hbm, zero_hbm, out_hbm,
                 src_v, dst_v, rows_v, agg_sh, sem):
    c = lax.axis_index("c")
    s = lax.axis_index("s")
    wid = c * _NS + s
    pltpu.sync_copy(src_hbm.at[wid], src_v)
    pltpu.sync_copy(dst_hbm.at[wid], dst_v)
    pltpu.sync_copy(zero_hbm, agg_sh.at[pl.ds(s * _RPS, _RPS)])
    plsc.subcore_barrier()

    def body(j, carry):
        pltpu.async_copy(h_hbm.at[src_v.at[j]], rows_v, sem).wait()
        pltpu.sync_copy(rows_v, agg_sh.at[dst_v.at[j]], add=True)
        return carry

    lax.fori_loop(0, _NCH, body, 0)
    plsc.subcore_barrier()
    pltpu.sync_copy(agg_sh.at[pl.ds(s * _RPS, _RPS)],
                    out_hbm.at[c].at[pl.ds(s * _RPS, _RPS)])


_sc_agg = pl.kernel(
    _sc_agg_body,
    out_type=jax.ShapeDtypeStruct((_NC, _NPAD, _D), jnp.float32),
    mesh=plsc.VectorSubcoreMesh(core_axis_name="c", subcore_axis_name="s",
                                num_cores=_NC, num_subcores=_NS),
    scratch_types=[
        pltpu.VMEM((_NCH, _CH), jnp.int32),
        pltpu.VMEM((_NCH, _CH), jnp.int32),
        pltpu.VMEM((_CH, _D), jnp.float32),
        pltpu.VMEM_SHARED((_NPAD, _D), jnp.float32),
        pltpu.SemaphoreType.DMA,
    ],
)


def _mm_stats_body(eps_ref, h_ref, a_ref, w_ref, u_ref, s_ref):
    i = pl.program_id(0)
    z = h_ref[...] * eps_ref[0, 0] + a_ref[0] + a_ref[1]
    u = jnp.dot(z, w_ref[...], preferred_element_type=jnp.float32)
    u_ref[...] = u
    st = jnp.concatenate(
        [jnp.sum(u, axis=0, keepdims=True),
         jnp.sum(u * u, axis=0, keepdims=True)], axis=0)

    @pl.when(i == 0)
    def _():
        s_ref[...] = st

    @pl.when(i > 0)
    def _():
        s_ref[...] += st


def _mm_stats(epsb, h, agg, wt):
    return pl.pallas_call(
        _mm_stats_body,
        grid=(_N // _BLK,),
        in_specs=[
            pl.BlockSpec(memory_space=pltpu.SMEM),
            pl.BlockSpec((_BLK, _D), lambda i: (i, 0)),
            pl.BlockSpec((_NC, _BLK, _D), lambda i: (0, i, 0)),
            pl.BlockSpec((_D, _D), lambda i: (0, 0)),
        ],
        out_specs=[
            pl.BlockSpec((_BLK, _D), lambda i: (i, 0)),
            pl.BlockSpec((2, _D), lambda i: (0, 0)),
        ],
        out_shape=[
            jax.ShapeDtypeStruct((_N, _D), jnp.float32),
            jax.ShapeDtypeStruct((2, _D), jnp.float32),
        ],
    )(epsb, h, agg, wt)


def _bn_coeffs(s_ref, g_ref, b_ref):
    mean = s_ref[0:1, :] * (1.0 / _N)
    var = s_ref[1:2, :] * (1.0 / _N) - mean * mean
    inv = lax.rsqrt(var + 1e-5)
    scale = g_ref[...] * inv
    shift = b_ref[...] - mean * scale
    return scale, shift


def _bn_relu_mm_body(s_ref, g_ref, b_ref, u_ref, w_ref, v_ref, s2_ref):
    i = pl.program_id(0)
    scale, shift = _bn_coeffs(s_ref, g_ref, b_ref)
    r = jnp.maximum(u_ref[...] * scale + shift, 0.0)
    v = jnp.dot(r, w_ref[...], preferred_element_type=jnp.float32)
    v_ref[...] = v
    st = jnp.concatenate(
        [jnp.sum(v, axis=0, keepdims=True),
         jnp.sum(v * v, axis=0, keepdims=True)], axis=0)

    @pl.when(i == 0)
    def _():
        s2_ref[...] = st

    @pl.when(i > 0)
    def _():
        s2_ref[...] += st


def _bn_relu_mm(s1, g, b, u, wt):
    return pl.pallas_call(
        _bn_relu_mm_body,
        grid=(_N // _BLK,),
        in_specs=[
            pl.BlockSpec((2, _D), lambda i: (0, 0)),
            pl.BlockSpec((1, _D), lambda i: (0, 0)),
            pl.BlockSpec((1, _D), lambda i: (0, 0)),
            pl.BlockSpec((_BLK, _D), lambda i: (i, 0)),
            pl.BlockSpec((_D, _D), lambda i: (0, 0)),
        ],
        out_specs=[
            pl.BlockSpec((_BLK, _D), lambda i: (i, 0)),
            pl.BlockSpec((2, _D), lambda i: (0, 0)),
        ],
        out_shape=[
            jax.ShapeDtypeStruct((_N, _D), jnp.float32),
            jax.ShapeDtypeStruct((2, _D), jnp.float32),
        ],
    )(s1, g, b, u, wt)


def _bn_relu_body(s_ref, g_ref, b_ref, v_ref, h_ref):
    scale, shift = _bn_coeffs(s_ref, g_ref, b_ref)
    h_ref[...] = jnp.maximum(v_ref[...] * scale + shift, 0.0)


def _bn_relu(s2, g, b, v):
    return pl.pallas_call(
        _bn_relu_body,
        grid=(_N // _BLK,),
        in_specs=[
            pl.BlockSpec((2, _D), lambda i: (0, 0)),
            pl.BlockSpec((1, _D), lambda i: (0, 0)),
            pl.BlockSpec((1, _D), lambda i: (0, 0)),
            pl.BlockSpec((_BLK, _D), lambda i: (i, 0)),
        ],
        out_specs=pl.BlockSpec((_BLK, _D), lambda i: (i, 0)),
        out_shape=jax.ShapeDtypeStruct((_N, _D), jnp.float32),
    )(s2, g, b, v)


def _readout_body(s_ref, g_ref, b_ref, bat_ref, v_ref, wm1_ref, bm1_ref,
                  wm2_ref, bm2_ref, lg_ref, ge_ref, ne_ref,
                  sums_ref, cnts_ref):
    i = pl.program_id(0)

    @pl.when(i == 0)
    def _():
        sums_ref[...] = jnp.zeros_like(sums_ref)
        cnts_ref[...] = jnp.zeros_like(cnts_ref)

    scale, shift = _bn_coeffs(s_ref, g_ref, b_ref)
    h = jnp.maximum(v_ref[...] * scale + shift, 0.0)
    ne_ref[...] = h
    b = bat_ref[0]
    ids = lax.broadcasted_iota(jnp.int32, (_G, _BLK), 0)
    oh = (b == ids).astype(jnp.float32)
    sums_ref[...] += jnp.dot(oh, h, preferred_element_type=jnp.float32)
    cnts_ref[...] += jnp.broadcast_to(
        jnp.sum(oh, axis=1, keepdims=True), (_G, _D))

    @pl.when(i == pl.num_programs(0) - 1)
    def _():
        ge = sums_ref[...] / jnp.maximum(cnts_ref[...], 1.0)
        ge_ref[...] = ge
        z = jnp.dot(ge, wm1_ref[...], preferred_element_type=jnp.float32)
        z = z + bm1_ref[...]
        z = jnp.where(z > 0.0, z, jnp.exp(jnp.minimum(z, 0.0)) - 1.0)
        lg_ref[...] = (jnp.dot(z, wm2_ref[...],
                               preferred_element_type=jnp.float32)
                       + bm2_ref[...])


def _readout(s2, g, b, bat3, v, wm1t, bm1r, wm2tp, bm2p):
    return pl.pallas_call(
        _readout_body,
        grid=(_N // _BLK,),
        in_specs=[
            pl.BlockSpec((2, _D), lambda i: (0, 0)),
            pl.BlockSpec((1, _D), lambda i: (0, 0)),
            pl.BlockSpec((1, _D), lambda i: (0, 0)),
            pl.BlockSpec((1, 1, _BLK), lambda i: (i, 0, 0)),
            pl.BlockSpec((_BLK, _D), lambda i: (i, 0)),
            pl.BlockSpec((_D, _D), lambda i: (0, 0)),
            pl.BlockSpec((1, _D), lambda i: (0, 0)),
            pl.BlockSpec((_D, _D), lambda i: (0, 0)),
            pl.BlockSpec((1, _D), lambda i: (0, 0)),
        ],
        out_specs=[
            pl.BlockSpec((_G, _D), lambda i: (0, 0)),
            pl.BlockSpec((_G, _D), lambda i: (0, 0)),
            pl.BlockSpec((_BLK, _D), lambda i: (i, 0)),
        ],
        out_shape=[
            jax.ShapeDtypeStruct((_G, _D), jnp.float32),
            jax.ShapeDtypeStruct((_G, _D), jnp.float32),
            jax.ShapeDtypeStruct((_N, _D), jnp.float32),
        ],
        scratch_shapes=[
            pltpu.VMEM((_G, _D), jnp.float32),
            pltpu.VMEM((_G, _D), jnp.float32),
        ],
    )(s2, g, b, bat3, v, wm1t, bm1r, wm2tp, bm2p)


def kernel(x, edge_index, batch,
           eps1, W1a, g1a, b1a, W1b, g1b, b1b,
           eps2, W2a, g2a, b2a, W2b, g2b, b2b,
           eps3, W3a, g3a, b3a, W3b, g3b, b3b,
           Wm1, bm1, Wm2, bm2):
    src_w = edge_index[0].reshape(_NW, _EPW)
    dst_w = edge_index[1].reshape(_NW, _EPW)
    pad = _EPW_PAD - _EPW
    src3 = jnp.pad(src_w, ((0, 0), (0, pad))).reshape(_NW, _NCH, _CH)
    dst3 = jnp.pad(dst_w, ((0, 0), (0, pad)),
                   constant_values=_N).reshape(_NW, _NCH, _CH)
    zero_rows = jnp.zeros((_RPS, _D), jnp.float32)

    h = x
    for (eps, Wa, ga, ba, Wb, gb, bb) in (
            (eps1, W1a, g1a, b1a, W1b, g1b, b1b),
            (eps2, W2a, g2a, b2a, W2b, g2b, b2b),
            (eps3, W3a, g3a, b3a, W3b, g3b, b3b)):
        agg = _sc_agg(h, src3, dst3, zero_rows)
        epsb = jnp.reshape(1.0 + eps, (1, 1))
        u, s1 = _mm_stats(epsb, h, agg, Wa.T)
        v, s2 = _bn_relu_mm(s1, ga.reshape(1, _D), ba.reshape(1, _D), u, Wb.T)
        if gb is not g3b:
            h = _bn_relu(s2, gb.reshape(1, _D), bb.reshape(1, _D), v)

    bat3 = batch.reshape(_N // _BLK, 1, _BLK)
    wm2tp = jnp.zeros((_D, _D), jnp.float32).at[:, :_OUT].set(Wm2.T)
    bm2p = jnp.zeros((1, _D), jnp.float32).at[0, :_OUT].set(bm2)
    logits_f, graph_emb, node_emb = _readout(
        s2, g3b.reshape(1, _D), b3b.reshape(1, _D), bat3, v, Wm1.T,
        bm1.reshape(1, _D), wm2tp, bm2p)
    return (logits_f[:, :_OUT], graph_emb, node_emb)
